# baseline (speedup 1.0000x reference)
import dataclasses
import functools

import jax
import jax.numpy as jnp
from jax import lax
from jax.experimental import pallas as pl
from jax.experimental.pallas import tpu as pltpu
from jax.experimental.pallas import tpu_sc as plsc

NUM_USER = 4000
NUM_ITEM = 6000
N = NUM_USER + NUM_ITEM
D = 256
E_HALF = 80000
E = 2 * E_HALF

NC = 2
NSUB = 16
NW = NC * NSUB
U_RPT = 250
I_RPT = 375
ACC_R = 384
TRASH = 376
CAP = 7680
KB = 64
NCHUNK = CAP // KB
WIN = 2000
NWIN = E // WIN
CWIN = 512
HIST = ACC_R * 16

KU = 32
UPT = 128
UWIN = 32

_MESH = plsc.VectorSubcoreMesh(core_axis_name="c", subcore_axis_name="s")

_CP = pltpu.CompilerParams()
if "needs_layout_passes" in pltpu.CompilerParams.__dataclass_fields__:
    _CP = dataclasses.replace(_CP, needs_layout_passes=False)


def _mlp_body(f_ref, w1_ref, b1_ref, w2_ref, b2_ref, o_ref):
    h = lax.dot_general(f_ref[...], w1_ref[...], (((1,), (1,)), ((), ())),
                        preferred_element_type=jnp.float32)
    h = h + b1_ref[...][None, :]
    h = jnp.where(h >= 0, h, 0.01 * h)
    o = lax.dot_general(h, w2_ref[...], (((1,), (1,)), ((), ())),
                        preferred_element_type=jnp.float32)
    o_ref[...] = o + b2_ref[...][None, :]


def _mlp(features, W1, b1, W2, b2, block=600):
    n, k = features.shape
    h_dim = W1.shape[0]
    grid = (n + block - 1) // block
    return pl.pallas_call(
        _mlp_body,
        grid=(grid,),
        in_specs=[
            pl.BlockSpec((block, k), lambda i: (i, 0)),
            pl.BlockSpec((h_dim, k), lambda i: (0, 0)),
            pl.BlockSpec((h_dim,), lambda i: (0,)),
            pl.BlockSpec((D, h_dim), lambda i: (0, 0)),
            pl.BlockSpec((D,), lambda i: (0,)),
        ],
        out_specs=pl.BlockSpec((block, D), lambda i: (i, 0)),
        out_shape=jax.ShapeDtypeStruct((n, D), jnp.float32),
    )(features, W1, b1, W2, b2)


def _wid():
    return lax.axis_index("s") * NC + lax.axis_index("c")


def _prep_body(row_hbm, col_hbm, listr_hbm, listc_hbm, cnt_hbm, hist_hbm,
               cb0, cb1, rb0, rb1, listr_v, listc_v, hist_v, cnt_v,
               g0, g1):
    wid = _wid()
    base = jnp.where(wid < 16, U_RPT * wid, NUM_USER + I_RPT * (wid - 16))
    rpt = jnp.where(wid < 16, U_RPT, I_RPT)
    iota = lax.iota(jnp.int32, 16)
    ones = jnp.ones((16,), jnp.float32)

    @pl.loop(0, CAP, step=16)
    def _(o):
        listr_v[pl.ds(o, 16)] = jnp.zeros((16,), jnp.int32)
        listc_v[pl.ds(o, 16)] = jnp.full((16,), TRASH, jnp.int32)

    @pl.loop(0, HIST, step=16)
    def _(o):
        hist_v[pl.ds(o, 16)] = jnp.zeros((16,), jnp.float32)

    pltpu.async_copy(col_hbm.at[pl.ds(0, WIN)], cb0, g0)
    pltpu.async_copy(row_hbm.at[pl.ds(0, WIN)], rb0, g0)
    pltpu.async_copy(col_hbm.at[pl.ds(WIN, WIN)], cb1, g1)
    pltpu.async_copy(row_hbm.at[pl.ds(WIN, WIN)], rb1, g1)

    def scan_window(cb, rb, ptr):
        def chunk(k, ptr):
            colv = cb[pl.ds(k * 16, 16)]
            rowv = rb[pl.ds(k * 16, 16)]
            local = colv - base
            mask = (local >= 0) & (local < rpt)
            lsafe = jnp.where(mask, local, TRASH)
            plsc.addupdate_scatter(hist_v, [lsafe * 16 + iota], ones)
            pc = plsc.cumsum(mask.astype(jnp.int32))
            idxs = jnp.minimum(ptr + pc - 1, CAP - 1)
            plsc.store_scatter(listr_v, [idxs], rowv, mask=mask)
            plsc.store_scatter(listc_v, [idxs], local, mask=mask)
            return ptr + pc[15]

        return lax.fori_loop(0, WIN // 16, chunk, ptr)

    def window_pair(w, ptr):
        pltpu.make_async_copy(col_hbm.at[pl.ds(0, WIN)], cb0, g0).wait()
        pltpu.make_async_copy(row_hbm.at[pl.ds(0, WIN)], rb0, g0).wait()
        ptr = scan_window(cb0, rb0, ptr)

        @pl.when(2 * w + 2 < NWIN)
        def _():
            pltpu.async_copy(col_hbm.at[pl.ds((2 * w + 2) * WIN, WIN)], cb0,
                             g0)
            pltpu.async_copy(row_hbm.at[pl.ds((2 * w + 2) * WIN, WIN)], rb0,
                             g0)

        pltpu.make_async_copy(col_hbm.at[pl.ds(0, WIN)], cb1, g1).wait()
        pltpu.make_async_copy(row_hbm.at[pl.ds(0, WIN)], rb1, g1).wait()
        ptr = scan_window(cb1, rb1, ptr)

        @pl.when(2 * w + 3 < NWIN)
        def _():
            pltpu.async_copy(col_hbm.at[pl.ds((2 * w + 3) * WIN, WIN)], cb1,
                             g1)
            pltpu.async_copy(row_hbm.at[pl.ds((2 * w + 3) * WIN, WIN)], rb1,
                             g1)

        return ptr

    ptr = lax.fori_loop(0, NWIN // 2, window_pair, jnp.int32(0))

    cnt_v[...] = jnp.broadcast_to(ptr, (16,)).astype(jnp.int32)
    pltpu.sync_copy(listr_v, listr_hbm.at[pl.ds(wid * CAP, CAP)])
    pltpu.sync_copy(listc_v, listc_hbm.at[pl.ds(wid * CAP, CAP)])
    pltpu.sync_copy(cnt_v, cnt_hbm.at[wid])
    pltpu.sync_copy(hist_v, hist_hbm.at[wid])


def _sc_prep(row, col):
    kern = pl.kernel(
        _prep_body,
        mesh=_MESH,
        compiler_params=_CP,
        out_type=(
            jax.ShapeDtypeStruct((NW * CAP,), jnp.int32),
            jax.ShapeDtypeStruct((NW * CAP,), jnp.int32),
            jax.ShapeDtypeStruct((NW, 16), jnp.int32),
            jax.ShapeDtypeStruct((NW, HIST), jnp.float32),
        ),
        scratch_types=[
            pltpu.VMEM((WIN,), jnp.int32),
            pltpu.VMEM((WIN,), jnp.int32),
            pltpu.VMEM((WIN,), jnp.int32),
            pltpu.VMEM((WIN,), jnp.int32),
            pltpu.VMEM((CAP,), jnp.int32),
            pltpu.VMEM((CAP,), jnp.int32),
            pltpu.VMEM((HIST,), jnp.float32),
            pltpu.VMEM((16,), jnp.int32),
            pltpu.SemaphoreType.DMA,
            pltpu.SemaphoreType.DMA,
        ],
    )
    return kern(row, col)


def _conv_body(y_hbm, listr_hbm, listc_hbm, cnt_hbm, zeros_hbm, out_hbm,
               listr_v, listc_v, cnt_v, idxb0, idxb1, buf0, buf1, acc_v,
               g0, g1):
    wid = _wid()
    pltpu.sync_copy(zeros_hbm, acc_v)
    pltpu.sync_copy(listr_hbm.at[pl.ds(wid * CAP, CAP)], listr_v)
    pltpu.sync_copy(listc_hbm.at[pl.ds(wid * CAP, CAP)], listc_v)
    pltpu.sync_copy(cnt_hbm.at[wid], cnt_v)
    n = cnt_v[pl.ds(0, 16)][0]
    nch = jnp.minimum((n + KB - 1) // KB + 1, NCHUNK)

    def fill_idx(dst, j):
        for o in range(0, KB, 16):
            dst[pl.ds(o, 16)] = listr_v[pl.ds(j * KB + o, 16)]

    def consume(buf, j):
        for eo in range(0, KB, 16):
            colv = listc_v[pl.ds(j * KB + eo, 16)]
            for e in range(16):
                cb = colv[e] * D
                for o in range(0, D // 2, 16):
                    bv = plsc.bitcast(buf[eo + e, pl.ds(o, 16)],
                                      jnp.bfloat16)
                    lo, hi = plsc.unpack(bv,
                                         format=plsc.PackFormat.INTERLEAVED)
                    plsc.addupdate(acc_v.at[pl.ds(cb + 2 * o, 16)], lo)
                    plsc.addupdate(acc_v.at[pl.ds(cb + 2 * o + 16, 16)], hi)

    fill_idx(idxb0, 0)
    pltpu.async_copy(y_hbm.at[idxb0], buf0, g0)
    fill_idx(idxb1, 1)
    pltpu.async_copy(y_hbm.at[idxb1], buf1, g1)

    @pl.loop(0, NCHUNK, step=2)
    def _(j):
        @pl.when(j < nch)
        def _():
            pltpu.make_async_copy(y_hbm.at[idxb0], buf0, g0).wait()
            consume(buf0, j)

            @pl.when(j + 2 < nch)
            def _():
                fill_idx(idxb0, j + 2)
                pltpu.async_copy(y_hbm.at[idxb0], buf0, g0)

        @pl.when(j + 1 < nch)
        def _():
            pltpu.make_async_copy(y_hbm.at[idxb1], buf1, g1).wait()
            consume(buf1, j + 1)

            @pl.when(j + 3 < nch)
            def _():
                fill_idx(idxb1, j + 3)
                pltpu.async_copy(y_hbm.at[idxb1], buf1, g1)

    @pl.when(wid < 16)
    def _():
        pltpu.sync_copy(acc_v.at[pl.ds(0, U_RPT * D)],
                        out_hbm.at[pl.ds(wid * U_RPT * D, U_RPT * D)])

    @pl.when(wid >= 16)
    def _():
        pltpu.sync_copy(
            acc_v.at[pl.ds(0, I_RPT * D)],
            out_hbm.at[pl.ds((NUM_USER + (wid - 16) * I_RPT) * D,
                             I_RPT * D)])


def _sc_conv(y, listr, listc, cnt, zeros_acc):
    kern = pl.kernel(
        _conv_body,
        mesh=_MESH,
        compiler_params=_CP,
        out_type=jax.ShapeDtypeStruct((N * D,), jnp.float32),
        scratch_types=[
            pltpu.VMEM((CAP,), jnp.int32),
            pltpu.VMEM((CAP,), jnp.int32),
            pltpu.VMEM((16,), jnp.int32),
            pltpu.VMEM((KB,), jnp.int32),
            pltpu.VMEM((KB,), jnp.int32),
            pltpu.VMEM((KB, D // 2), jnp.int32),
            pltpu.VMEM((KB, D // 2), jnp.int32),
            pltpu.VMEM((ACC_R * D,), jnp.float32),
            pltpu.SemaphoreType.DMA,
            pltpu.SemaphoreType.DMA,
        ],
    )
    return kern(y, listr, listc, cnt, zeros_acc)


def _ugraph_body(rep_hbm, ug_hbm, w_hbm, out_hbm, ug_v, w_v, idxb0, idxb1,
                 buf0, buf1, out_v, g0, g1):
    wid = _wid()
    pltpu.sync_copy(ug_hbm.at[pl.ds(wid * UPT * KU, UPT * KU)], ug_v)
    pltpu.sync_copy(w_hbm.at[pl.ds(wid * UPT * KU, UPT * KU)], w_v)

    def fill_idx(dst, u):
        for o in range(0, KU, 16):
            dst[pl.ds(o, 16)] = ug_v[pl.ds(u * KU + o, 16)]

    def consume(buf, uu):
        wvs = [w_v[pl.ds(uu * KU + ko, 16)] for ko in range(0, KU, 16)]
        for o in range(0, D, 16):
            acc = jnp.zeros((16,), jnp.float32)
            for k in range(KU):
                acc = acc + wvs[k // 16][k % 16] * buf[k, pl.ds(o, 16)]
            out_v[uu, pl.ds(o, 16)] = acc

    fill_idx(idxb0, 0)
    pltpu.async_copy(rep_hbm.at[idxb0], buf0, g0)
    fill_idx(idxb1, 1)
    pltpu.async_copy(rep_hbm.at[idxb1], buf1, g1)

    @pl.loop(0, UPT, step=2)
    def _(u):
        pltpu.make_async_copy(rep_hbm.at[idxb0], buf0, g0).wait()
        consume(buf0, u)

        @pl.when(u + 2 < UPT)
        def _():
            fill_idx(idxb0, u + 2)
            pltpu.async_copy(rep_hbm.at[idxb0], buf0, g0)

        pltpu.make_async_copy(rep_hbm.at[idxb1], buf1, g1).wait()
        consume(buf1, u + 1)

        @pl.when(u + 3 < UPT)
        def _():
            fill_idx(idxb1, u + 3)
            pltpu.async_copy(rep_hbm.at[idxb1], buf1, g1)

    pltpu.sync_copy(out_v, out_hbm.at[pl.ds(wid * UPT, UPT)])


def _sc_ugraph(user_rep, ug_flat, w_flat):
    kern = pl.kernel(
        _ugraph_body,
        mesh=_MESH,
        compiler_params=_CP,
        out_type=jax.ShapeDtypeStruct((NW * UPT, D), jnp.float32),
        scratch_types=[
            pltpu.VMEM((UPT * KU,), jnp.int32),
            pltpu.VMEM((UPT * KU,), jnp.float32),
            pltpu.VMEM((KU,), jnp.int32),
            pltpu.VMEM((KU,), jnp.int32),
            pltpu.VMEM((KU, D), jnp.float32),
            pltpu.VMEM((KU, D), jnp.float32),
            pltpu.VMEM((UPT, D), jnp.float32),
            pltpu.SemaphoreType.DMA,
            pltpu.SemaphoreType.DMA,
        ],
    )
    return kern(user_rep, ug_flat, w_flat)


def _ileave_bf16(y):
    yb = y.reshape(N, 8, 2, 16).swapaxes(2, 3).reshape(N, D // 2, 2) \
        .astype(jnp.bfloat16)
    return lax.bitcast_convert_type(yb, jnp.int32)


def _branch(temp, preference, lists, dis, zeros_acc):
    listr, listc, cnt = lists
    x = jnp.concatenate([preference, temp], axis=0)
    x = x / jnp.maximum(jnp.linalg.norm(x, axis=1, keepdims=True), 1e-12)
    disc = dis[:, None]
    y1 = x * disc
    s1 = _sc_conv(_ileave_bf16(y1), listr, listc, cnt,
                  zeros_acc).reshape(N, D)
    y2 = s1 * (disc * disc)
    s2 = _sc_conv(_ileave_bf16(y2), listr, listc, cnt,
                  zeros_acc).reshape(N, D)
    return x + (s1 + s2) * disc


def kernel(edge_index, v_feat, t_feat, pref_v, pref_t, W1v, b1v, W2v, b2v,
           W1t, b1t, W2t, b2t, weight_u, user_graph, user_weight_matrix):
    row = edge_index[0].astype(jnp.int32)
    col = edge_index[1].astype(jnp.int32)

    listr, listc, cnt, hist = _sc_prep(row, col)
    lists = (listr, listc, cnt)
    hsum = hist.reshape(NW, ACC_R, 16).sum(axis=2)
    deg = jnp.concatenate([hsum[:16, :U_RPT].reshape(NUM_USER),
                           hsum[16:, :I_RPT].reshape(NUM_ITEM)])
    dis = jnp.where(deg > 0, lax.rsqrt(jnp.maximum(deg, 1e-30)), 0.0)
    zeros_acc = jnp.zeros((ACC_R * D,), jnp.float32)

    temp_v = _mlp(v_feat, W1v, b1v, W2v, b2v)
    temp_t = _mlp(t_feat, W1t, b1t, W2t, b2t)
    v_rep = _branch(temp_v, pref_v, lists, dis, zeros_acc)
    t_rep = _branch(temp_t, pref_t, lists, dis, zeros_acc)
    representation = v_rep + t_rep
    user_stack = jnp.stack([v_rep[:NUM_USER], t_rep[:NUM_USER]], axis=2)
    user_rep = jnp.squeeze(user_stack @ weight_u, axis=2)
    item_rep = representation[NUM_USER:]

    ug = user_graph.astype(jnp.int32)
    ug = jnp.concatenate(
        [ug, jnp.zeros((NUM_USER, KU - ug.shape[1]), jnp.int32)], axis=1)
    ug = jnp.concatenate(
        [ug, jnp.zeros((NW * UPT - NUM_USER, KU), jnp.int32)], axis=0)
    w = jnp.concatenate(
        [user_weight_matrix,
         jnp.zeros((NUM_USER, KU - user_weight_matrix.shape[1]),
                   jnp.float32)], axis=1)
    w = jnp.concatenate(
        [w, jnp.zeros((NW * UPT - NUM_USER, KU), jnp.float32)], axis=0)
    h_u1 = _sc_ugraph(user_rep, ug.reshape(-1), w.reshape(-1))[:NUM_USER]
    user_rep = user_rep + h_u1
    return jnp.concatenate([user_rep, item_rep], axis=0)


# speedup vs baseline: 1.1117x; 1.1117x over previous
import dataclasses
import functools

import jax
import jax.numpy as jnp
from jax import lax
from jax.experimental import pallas as pl
from jax.experimental.pallas import tpu as pltpu
from jax.experimental.pallas import tpu_sc as plsc

NUM_USER = 4000
NUM_ITEM = 6000
N = NUM_USER + NUM_ITEM
D = 256
E_HALF = 80000
E = 2 * E_HALF

NC = 2
NSUB = 16
NW = NC * NSUB
U_RPT = 250
I_RPT = 375
ACC_R = 384
TRASH = 376
CAP = 7680
KB = 32
NCHUNK = CAP // KB
WIN = 2000
NWIN = E // WIN
CWIN = 512
HIST = ACC_R * 16

KU = 32
UPT = 128
UWIN = 32

_MESH = plsc.VectorSubcoreMesh(core_axis_name="c", subcore_axis_name="s")

_CP = pltpu.CompilerParams()
if "needs_layout_passes" in pltpu.CompilerParams.__dataclass_fields__:
    _CP = dataclasses.replace(_CP, needs_layout_passes=False)


def _mlp_body(f_ref, w1_ref, b1_ref, w2_ref, b2_ref, o_ref):
    h = lax.dot_general(f_ref[...], w1_ref[...], (((1,), (1,)), ((), ())),
                        preferred_element_type=jnp.float32)
    h = h + b1_ref[...][None, :]
    h = jnp.where(h >= 0, h, 0.01 * h)
    o = lax.dot_general(h, w2_ref[...], (((1,), (1,)), ((), ())),
                        preferred_element_type=jnp.float32)
    o_ref[...] = o + b2_ref[...][None, :]


def _mlp(features, W1, b1, W2, b2, block=600):
    n, k = features.shape
    h_dim = W1.shape[0]
    grid = (n + block - 1) // block
    return pl.pallas_call(
        _mlp_body,
        grid=(grid,),
        in_specs=[
            pl.BlockSpec((block, k), lambda i: (i, 0)),
            pl.BlockSpec((h_dim, k), lambda i: (0, 0)),
            pl.BlockSpec((h_dim,), lambda i: (0,)),
            pl.BlockSpec((D, h_dim), lambda i: (0, 0)),
            pl.BlockSpec((D,), lambda i: (0,)),
        ],
        out_specs=pl.BlockSpec((block, D), lambda i: (i, 0)),
        out_shape=jax.ShapeDtypeStruct((n, D), jnp.float32),
    )(features, W1, b1, W2, b2)


def _wid():
    return lax.axis_index("s") * NC + lax.axis_index("c")


def _prep_body(row_hbm, col_hbm, listr_hbm, listc_hbm, cnt_hbm, hist_hbm,
               cb0, cb1, rb0, rb1, listr_v, listc_v, hist_v, cnt_v,
               g0, g1):
    wid = _wid()
    base = jnp.where(wid < 16, U_RPT * wid, NUM_USER + I_RPT * (wid - 16))
    rpt = jnp.where(wid < 16, U_RPT, I_RPT)
    iota = lax.iota(jnp.int32, 16)
    ones = jnp.ones((16,), jnp.float32)

    @pl.loop(0, CAP, step=16)
    def _(o):
        listr_v[pl.ds(o, 16)] = jnp.zeros((16,), jnp.int32)
        listc_v[pl.ds(o, 16)] = jnp.full((16,), TRASH, jnp.int32)

    @pl.loop(0, HIST, step=16)
    def _(o):
        hist_v[pl.ds(o, 16)] = jnp.zeros((16,), jnp.float32)

    pltpu.async_copy(col_hbm.at[pl.ds(0, WIN)], cb0, g0)
    pltpu.async_copy(row_hbm.at[pl.ds(0, WIN)], rb0, g0)
    pltpu.async_copy(col_hbm.at[pl.ds(WIN, WIN)], cb1, g1)
    pltpu.async_copy(row_hbm.at[pl.ds(WIN, WIN)], rb1, g1)

    def scan_window(cb, rb, ptr):
        def chunk(k, ptr):
            colv = cb[pl.ds(k * 16, 16)]
            rowv = rb[pl.ds(k * 16, 16)]
            local = colv - base
            mask = (local >= 0) & (local < rpt)
            lsafe = jnp.where(mask, local, TRASH)
            plsc.addupdate_scatter(hist_v, [lsafe * 16 + iota], ones)
            pc = plsc.cumsum(mask.astype(jnp.int32))
            idxs = jnp.minimum(ptr + pc - 1, CAP - 1)
            plsc.store_scatter(listr_v, [idxs], rowv, mask=mask)
            plsc.store_scatter(listc_v, [idxs], local, mask=mask)
            return ptr + pc[15]

        return lax.fori_loop(0, WIN // 16, chunk, ptr)

    def window_pair(w, ptr):
        pltpu.make_async_copy(col_hbm.at[pl.ds(0, WIN)], cb0, g0).wait()
        pltpu.make_async_copy(row_hbm.at[pl.ds(0, WIN)], rb0, g0).wait()
        ptr = scan_window(cb0, rb0, ptr)

        @pl.when(2 * w + 2 < NWIN)
        def _():
            pltpu.async_copy(col_hbm.at[pl.ds((2 * w + 2) * WIN, WIN)], cb0,
                             g0)
            pltpu.async_copy(row_hbm.at[pl.ds((2 * w + 2) * WIN, WIN)], rb0,
                             g0)

        pltpu.make_async_copy(col_hbm.at[pl.ds(0, WIN)], cb1, g1).wait()
        pltpu.make_async_copy(row_hbm.at[pl.ds(0, WIN)], rb1, g1).wait()
        ptr = scan_window(cb1, rb1, ptr)

        @pl.when(2 * w + 3 < NWIN)
        def _():
            pltpu.async_copy(col_hbm.at[pl.ds((2 * w + 3) * WIN, WIN)], cb1,
                             g1)
            pltpu.async_copy(row_hbm.at[pl.ds((2 * w + 3) * WIN, WIN)], rb1,
                             g1)

        return ptr

    ptr = lax.fori_loop(0, NWIN // 2, window_pair, jnp.int32(0))

    cnt_v[...] = jnp.broadcast_to(ptr, (16,)).astype(jnp.int32)
    pltpu.sync_copy(listr_v, listr_hbm.at[pl.ds(wid * CAP, CAP)])
    pltpu.sync_copy(listc_v, listc_hbm.at[pl.ds(wid * CAP, CAP)])
    pltpu.sync_copy(cnt_v, cnt_hbm.at[wid])
    pltpu.sync_copy(hist_v, hist_hbm.at[wid])


def _sc_prep(row, col):
    kern = pl.kernel(
        _prep_body,
        mesh=_MESH,
        compiler_params=_CP,
        out_type=(
            jax.ShapeDtypeStruct((NW * CAP,), jnp.int32),
            jax.ShapeDtypeStruct((NW * CAP,), jnp.int32),
            jax.ShapeDtypeStruct((NW, 16), jnp.int32),
            jax.ShapeDtypeStruct((NW, HIST), jnp.float32),
        ),
        scratch_types=[
            pltpu.VMEM((WIN,), jnp.int32),
            pltpu.VMEM((WIN,), jnp.int32),
            pltpu.VMEM((WIN,), jnp.int32),
            pltpu.VMEM((WIN,), jnp.int32),
            pltpu.VMEM((CAP,), jnp.int32),
            pltpu.VMEM((CAP,), jnp.int32),
            pltpu.VMEM((HIST,), jnp.float32),
            pltpu.VMEM((16,), jnp.int32),
            pltpu.SemaphoreType.DMA,
            pltpu.SemaphoreType.DMA,
        ],
    )
    return kern(row, col)


def _conv_body(y_hbm, listr_hbm, listc_hbm, cnt_hbm, zeros_hbm, out_hbm,
               listr_v, listc_v, cnt_v, idxb0, idxb1, buf0, buf1, acc_v,
               g0, g1):
    wid = _wid()
    pltpu.sync_copy(zeros_hbm, acc_v)
    pltpu.sync_copy(listr_hbm.at[pl.ds(wid * CAP, CAP)], listr_v)
    pltpu.sync_copy(listc_hbm.at[pl.ds(wid * CAP, CAP)], listc_v)
    pltpu.sync_copy(cnt_hbm.at[wid], cnt_v)
    n = cnt_v[pl.ds(0, 16)][0]
    nch = jnp.minimum((n + KB - 1) // KB + 1, NCHUNK)

    def fill_idx(dst, j):
        for o in range(0, KB, 16):
            dst[pl.ds(o, 16)] = listr_v[pl.ds(j * KB + o, 16)]

    def consume(buf, j):
        for eo in range(0, KB, 16):
            colv = listc_v[pl.ds(j * KB + eo, 16)]
            for e in range(16):
                cb = colv[e] * D
                for o in range(0, D // 2, 16):
                    bv = plsc.bitcast(buf[eo + e, pl.ds(o, 16)],
                                      jnp.bfloat16)
                    lo, hi = plsc.unpack(bv,
                                         format=plsc.PackFormat.INTERLEAVED)
                    plsc.addupdate(acc_v.at[pl.ds(cb + 2 * o, 16)], lo)
                    plsc.addupdate(acc_v.at[pl.ds(cb + 2 * o + 16, 16)], hi)

    fill_idx(idxb0, 0)
    pltpu.async_copy(y_hbm.at[idxb0], buf0, g0)
    fill_idx(idxb1, 1)
    pltpu.async_copy(y_hbm.at[idxb1], buf1, g1)

    @pl.loop(0, NCHUNK, step=2)
    def _(j):
        @pl.when(j < nch)
        def _():
            pltpu.make_async_copy(y_hbm.at[idxb0], buf0, g0).wait()
            consume(buf0, j)

            @pl.when(j + 2 < nch)
            def _():
                fill_idx(idxb0, j + 2)
                pltpu.async_copy(y_hbm.at[idxb0], buf0, g0)

        @pl.when(j + 1 < nch)
        def _():
            pltpu.make_async_copy(y_hbm.at[idxb1], buf1, g1).wait()
            consume(buf1, j + 1)

            @pl.when(j + 3 < nch)
            def _():
                fill_idx(idxb1, j + 3)
                pltpu.async_copy(y_hbm.at[idxb1], buf1, g1)

    @pl.when(wid < 16)
    def _():
        pltpu.sync_copy(acc_v.at[pl.ds(0, U_RPT * D)],
                        out_hbm.at[pl.ds(wid * U_RPT * D, U_RPT * D)])

    @pl.when(wid >= 16)
    def _():
        pltpu.sync_copy(
            acc_v.at[pl.ds(0, I_RPT * D)],
            out_hbm.at[pl.ds((NUM_USER + (wid - 16) * I_RPT) * D,
                             I_RPT * D)])


def _sc_conv(y, listr, listc, cnt, zeros_acc):
    kern = pl.kernel(
        _conv_body,
        mesh=_MESH,
        compiler_params=_CP,
        out_type=jax.ShapeDtypeStruct((N * D,), jnp.float32),
        scratch_types=[
            pltpu.VMEM((CAP,), jnp.int32),
            pltpu.VMEM((CAP,), jnp.int32),
            pltpu.VMEM((16,), jnp.int32),
            pltpu.VMEM((KB,), jnp.int32),
            pltpu.VMEM((KB,), jnp.int32),
            pltpu.VMEM((KB, D // 2), jnp.int32),
            pltpu.VMEM((KB, D // 2), jnp.int32),
            pltpu.VMEM((ACC_R * D,), jnp.float32),
            pltpu.SemaphoreType.DMA,
            pltpu.SemaphoreType.DMA,
        ],
    )
    return kern(y, listr, listc, cnt, zeros_acc)


def _ugraph_body(rep_hbm, ug_hbm, w_hbm, out_hbm, ug_v, w_v, idxb0, idxb1,
                 buf0, buf1, out_v, g0, g1):
    wid = _wid()
    pltpu.sync_copy(ug_hbm.at[pl.ds(wid * UPT * KU, UPT * KU)], ug_v)
    pltpu.sync_copy(w_hbm.at[pl.ds(wid * UPT * KU, UPT * KU)], w_v)

    def fill_idx(dst, u):
        for o in range(0, KU, 16):
            dst[pl.ds(o, 16)] = ug_v[pl.ds(u * KU + o, 16)]

    def consume(buf, uu):
        wvs = [w_v[pl.ds(uu * KU + ko, 16)] for ko in range(0, KU, 16)]
        for o in range(0, D, 16):
            acc = jnp.zeros((16,), jnp.float32)
            for k in range(KU):
                acc = acc + wvs[k // 16][k % 16] * buf[k, pl.ds(o, 16)]
            out_v[uu, pl.ds(o, 16)] = acc

    fill_idx(idxb0, 0)
    pltpu.async_copy(rep_hbm.at[idxb0], buf0, g0)
    fill_idx(idxb1, 1)
    pltpu.async_copy(rep_hbm.at[idxb1], buf1, g1)

    @pl.loop(0, UPT, step=2)
    def _(u):
        pltpu.make_async_copy(rep_hbm.at[idxb0], buf0, g0).wait()
        consume(buf0, u)

        @pl.when(u + 2 < UPT)
        def _():
            fill_idx(idxb0, u + 2)
            pltpu.async_copy(rep_hbm.at[idxb0], buf0, g0)

        pltpu.make_async_copy(rep_hbm.at[idxb1], buf1, g1).wait()
        consume(buf1, u + 1)

        @pl.when(u + 3 < UPT)
        def _():
            fill_idx(idxb1, u + 3)
            pltpu.async_copy(rep_hbm.at[idxb1], buf1, g1)

    pltpu.sync_copy(out_v, out_hbm.at[pl.ds(wid * UPT, UPT)])


def _sc_ugraph(user_rep, ug_flat, w_flat):
    kern = pl.kernel(
        _ugraph_body,
        mesh=_MESH,
        compiler_params=_CP,
        out_type=jax.ShapeDtypeStruct((NW * UPT, D), jnp.float32),
        scratch_types=[
            pltpu.VMEM((UPT * KU,), jnp.int32),
            pltpu.VMEM((UPT * KU,), jnp.float32),
            pltpu.VMEM((KU,), jnp.int32),
            pltpu.VMEM((KU,), jnp.int32),
            pltpu.VMEM((KU, D), jnp.float32),
            pltpu.VMEM((KU, D), jnp.float32),
            pltpu.VMEM((UPT, D), jnp.float32),
            pltpu.SemaphoreType.DMA,
            pltpu.SemaphoreType.DMA,
        ],
    )
    return kern(user_rep, ug_flat, w_flat)


def _ileave_bf16(y):
    yb = y.reshape(N, 8, 2, 16).swapaxes(2, 3).reshape(N, D // 2, 2) \
        .astype(jnp.bfloat16)
    return lax.bitcast_convert_type(yb, jnp.int32)


def _branch(temp, preference, lists, dis, zeros_acc):
    listr, listc, cnt = lists
    x = jnp.concatenate([preference, temp], axis=0)
    x = x / jnp.maximum(jnp.linalg.norm(x, axis=1, keepdims=True), 1e-12)
    disc = dis[:, None]
    y1 = x * disc
    s1 = _sc_conv(_ileave_bf16(y1), listr, listc, cnt,
                  zeros_acc).reshape(N, D)
    y2 = s1 * (disc * disc)
    s2 = _sc_conv(_ileave_bf16(y2), listr, listc, cnt,
                  zeros_acc).reshape(N, D)
    return x + (s1 + s2) * disc


def kernel(edge_index, v_feat, t_feat, pref_v, pref_t, W1v, b1v, W2v, b2v,
           W1t, b1t, W2t, b2t, weight_u, user_graph, user_weight_matrix):
    row = edge_index[0].astype(jnp.int32)
    col = edge_index[1].astype(jnp.int32)

    listr, listc, cnt, hist = _sc_prep(row, col)
    lists = (listr, listc, cnt)
    hsum = hist.reshape(NW, ACC_R, 16).sum(axis=2)
    deg = jnp.concatenate([hsum[:16, :U_RPT].reshape(NUM_USER),
                           hsum[16:, :I_RPT].reshape(NUM_ITEM)])
    dis = jnp.where(deg > 0, lax.rsqrt(jnp.maximum(deg, 1e-30)), 0.0)
    zeros_acc = jnp.zeros((ACC_R * D,), jnp.float32)

    temp_v = _mlp(v_feat, W1v, b1v, W2v, b2v)
    temp_t = _mlp(t_feat, W1t, b1t, W2t, b2t)
    v_rep = _branch(temp_v, pref_v, lists, dis, zeros_acc)
    t_rep = _branch(temp_t, pref_t, lists, dis, zeros_acc)
    representation = v_rep + t_rep
    user_stack = jnp.stack([v_rep[:NUM_USER], t_rep[:NUM_USER]], axis=2)
    user_rep = jnp.squeeze(user_stack @ weight_u, axis=2)
    item_rep = representation[NUM_USER:]

    ug = user_graph.astype(jnp.int32)
    ug = jnp.concatenate(
        [ug, jnp.zeros((NUM_USER, KU - ug.shape[1]), jnp.int32)], axis=1)
    ug = jnp.concatenate(
        [ug, jnp.zeros((NW * UPT - NUM_USER, KU), jnp.int32)], axis=0)
    w = jnp.concatenate(
        [user_weight_matrix,
         jnp.zeros((NUM_USER, KU - user_weight_matrix.shape[1]),
                   jnp.float32)], axis=1)
    w = jnp.concatenate(
        [w, jnp.zeros((NW * UPT - NUM_USER, KU), jnp.float32)], axis=0)
    h_u1 = _sc_ugraph(user_rep, ug.reshape(-1), w.reshape(-1))[:NUM_USER]
    user_rep = user_rep + h_u1
    return jnp.concatenate([user_rep, item_rep], axis=0)


# speedup vs baseline: 1.1293x; 1.0158x over previous
import dataclasses
import functools

import jax
import jax.numpy as jnp
from jax import lax
from jax.experimental import pallas as pl
from jax.experimental.pallas import tpu as pltpu
from jax.experimental.pallas import tpu_sc as plsc

NUM_USER = 4000
NUM_ITEM = 6000
N = NUM_USER + NUM_ITEM
D = 256
E_HALF = 80000
E = 2 * E_HALF

NC = 2
NSUB = 16
NW = NC * NSUB
U_RPT = 250
I_RPT = 375
ACC_R = 384
TRASH = 376
CAP = 7680
KB = 32
NCHUNK = CAP // KB
WIN = 2000
NWIN = E // WIN
CWIN = 512
HIST = ACC_R * 16

KU = 32
UPT = 128
UWIN = 32

_MESH = plsc.VectorSubcoreMesh(core_axis_name="c", subcore_axis_name="s")

_CP = pltpu.CompilerParams()
if "needs_layout_passes" in pltpu.CompilerParams.__dataclass_fields__:
    _CP = dataclasses.replace(_CP, needs_layout_passes=False)


def _mlp_body(f_ref, w1_ref, b1_ref, w2_ref, b2_ref, o_ref):
    h = lax.dot_general(f_ref[...], w1_ref[...], (((1,), (1,)), ((), ())),
                        preferred_element_type=jnp.float32)
    h = h + b1_ref[...][None, :]
    h = jnp.where(h >= 0, h, 0.01 * h)
    o = lax.dot_general(h, w2_ref[...], (((1,), (1,)), ((), ())),
                        preferred_element_type=jnp.float32)
    o_ref[...] = o + b2_ref[...][None, :]


def _mlp(features, W1, b1, W2, b2, block=600):
    n, k = features.shape
    h_dim = W1.shape[0]
    grid = (n + block - 1) // block
    return pl.pallas_call(
        _mlp_body,
        grid=(grid,),
        in_specs=[
            pl.BlockSpec((block, k), lambda i: (i, 0)),
            pl.BlockSpec((h_dim, k), lambda i: (0, 0)),
            pl.BlockSpec((h_dim,), lambda i: (0,)),
            pl.BlockSpec((D, h_dim), lambda i: (0, 0)),
            pl.BlockSpec((D,), lambda i: (0,)),
        ],
        out_specs=pl.BlockSpec((block, D), lambda i: (i, 0)),
        out_shape=jax.ShapeDtypeStruct((n, D), jnp.float32),
    )(features, W1, b1, W2, b2)


def _wid():
    return lax.axis_index("s") * NC + lax.axis_index("c")


def _prep_body(row_hbm, col_hbm, listr_hbm, listc_hbm, cnt_hbm, hist_hbm,
               cb0, cb1, rb0, rb1, listr_v, listc_v, hist_v, cnt_v,
               g0, g1):
    wid = _wid()
    base = jnp.where(wid < 16, U_RPT * wid, NUM_USER + I_RPT * (wid - 16))
    rpt = jnp.where(wid < 16, U_RPT, I_RPT)
    iota = lax.iota(jnp.int32, 16)
    ones = jnp.ones((16,), jnp.float32)

    @pl.loop(0, CAP, step=16)
    def _(o):
        listr_v[pl.ds(o, 16)] = jnp.zeros((16,), jnp.int32)
        listc_v[pl.ds(o, 16)] = jnp.full((16,), TRASH, jnp.int32)

    @pl.loop(0, HIST, step=16)
    def _(o):
        hist_v[pl.ds(o, 16)] = jnp.zeros((16,), jnp.float32)

    pltpu.async_copy(col_hbm.at[pl.ds(0, WIN)], cb0, g0)
    pltpu.async_copy(row_hbm.at[pl.ds(0, WIN)], rb0, g0)
    pltpu.async_copy(col_hbm.at[pl.ds(WIN, WIN)], cb1, g1)
    pltpu.async_copy(row_hbm.at[pl.ds(WIN, WIN)], rb1, g1)

    def scan_window(cb, rb, ptr):
        def chunk(k, ptr):
            colv = cb[pl.ds(k * 16, 16)]
            rowv = rb[pl.ds(k * 16, 16)]
            local = colv - base
            mask = (local >= 0) & (local < rpt)
            lsafe = jnp.where(mask, local, TRASH)
            plsc.addupdate_scatter(hist_v, [lsafe * 16 + iota], ones)
            pc = plsc.cumsum(mask.astype(jnp.int32))
            idxs = jnp.minimum(ptr + pc - 1, CAP - 1)
            plsc.store_scatter(listr_v, [idxs], rowv, mask=mask)
            plsc.store_scatter(listc_v, [idxs], local, mask=mask)
            return ptr + pc[15]

        return lax.fori_loop(0, WIN // 16, chunk, ptr)

    def window_pair(w, ptr):
        pltpu.make_async_copy(col_hbm.at[pl.ds(0, WIN)], cb0, g0).wait()
        pltpu.make_async_copy(row_hbm.at[pl.ds(0, WIN)], rb0, g0).wait()
        ptr = scan_window(cb0, rb0, ptr)

        @pl.when(2 * w + 2 < NWIN)
        def _():
            pltpu.async_copy(col_hbm.at[pl.ds((2 * w + 2) * WIN, WIN)], cb0,
                             g0)
            pltpu.async_copy(row_hbm.at[pl.ds((2 * w + 2) * WIN, WIN)], rb0,
                             g0)

        pltpu.make_async_copy(col_hbm.at[pl.ds(0, WIN)], cb1, g1).wait()
        pltpu.make_async_copy(row_hbm.at[pl.ds(0, WIN)], rb1, g1).wait()
        ptr = scan_window(cb1, rb1, ptr)

        @pl.when(2 * w + 3 < NWIN)
        def _():
            pltpu.async_copy(col_hbm.at[pl.ds((2 * w + 3) * WIN, WIN)], cb1,
                             g1)
            pltpu.async_copy(row_hbm.at[pl.ds((2 * w + 3) * WIN, WIN)], rb1,
                             g1)

        return ptr

    ptr = lax.fori_loop(0, NWIN // 2, window_pair, jnp.int32(0))

    cnt_v[...] = jnp.broadcast_to(ptr, (16,)).astype(jnp.int32)
    pltpu.sync_copy(listr_v, listr_hbm.at[pl.ds(wid * CAP, CAP)])
    pltpu.sync_copy(listc_v, listc_hbm.at[pl.ds(wid * CAP, CAP)])
    pltpu.sync_copy(cnt_v, cnt_hbm.at[wid])
    pltpu.sync_copy(hist_v, hist_hbm.at[wid])


def _sc_prep(row, col):
    kern = pl.kernel(
        _prep_body,
        mesh=_MESH,
        compiler_params=_CP,
        out_type=(
            jax.ShapeDtypeStruct((NW * CAP,), jnp.int32),
            jax.ShapeDtypeStruct((NW * CAP,), jnp.int32),
            jax.ShapeDtypeStruct((NW, 16), jnp.int32),
            jax.ShapeDtypeStruct((NW, HIST), jnp.float32),
        ),
        scratch_types=[
            pltpu.VMEM((WIN,), jnp.int32),
            pltpu.VMEM((WIN,), jnp.int32),
            pltpu.VMEM((WIN,), jnp.int32),
            pltpu.VMEM((WIN,), jnp.int32),
            pltpu.VMEM((CAP,), jnp.int32),
            pltpu.VMEM((CAP,), jnp.int32),
            pltpu.VMEM((HIST,), jnp.float32),
            pltpu.VMEM((16,), jnp.int32),
            pltpu.SemaphoreType.DMA,
            pltpu.SemaphoreType.DMA,
        ],
    )
    return kern(row, col)


def _conv_body(y_hbm, listr_hbm, listc_hbm, cnt_hbm, zeros_hbm, out_hbm,
               listr_v, listc_v, cnt_v, idxb0, idxb1, buf0, buf1, acc_v,
               g0, g1):
    wid = _wid()
    pltpu.sync_copy(zeros_hbm, acc_v)
    pltpu.sync_copy(listr_hbm.at[pl.ds(wid * CAP, CAP)], listr_v)
    pltpu.sync_copy(listc_hbm.at[pl.ds(wid * CAP, CAP)], listc_v)
    pltpu.sync_copy(cnt_hbm.at[wid], cnt_v)
    n = cnt_v[pl.ds(0, 16)][0]
    nch = jnp.minimum((n + KB - 1) // KB + 1, NCHUNK)

    def fill_idx(dst, j):
        for o in range(0, KB, 16):
            dst[pl.ds(o, 16)] = listr_v[pl.ds(j * KB + o, 16)]

    def consume(buf, j):
        for eo in range(0, KB, 16):
            colv = listc_v[pl.ds(j * KB + eo, 16)]
            for e in range(16):
                cb = colv[e] * D
                for o in range(0, D // 2, 16):
                    bv = plsc.bitcast(buf[eo + e, pl.ds(o, 16)],
                                      jnp.bfloat16)
                    lo, hi = plsc.unpack(bv,
                                         format=plsc.PackFormat.INTERLEAVED)
                    plsc.addupdate(acc_v.at[pl.ds(cb + 2 * o, 16)], lo)
                    plsc.addupdate(acc_v.at[pl.ds(cb + 2 * o + 16, 16)], hi)

    fill_idx(idxb0, 0)
    pltpu.async_copy(y_hbm.at[idxb0], buf0, g0)
    fill_idx(idxb1, 1)
    pltpu.async_copy(y_hbm.at[idxb1], buf1, g1)

    @pl.loop(0, NCHUNK, step=2)
    def _(j):
        @pl.when(j < nch)
        def _():
            pltpu.make_async_copy(y_hbm.at[idxb0], buf0, g0).wait()
            consume(buf0, j)

            @pl.when(j + 2 < nch)
            def _():
                fill_idx(idxb0, j + 2)
                pltpu.async_copy(y_hbm.at[idxb0], buf0, g0)

        @pl.when(j + 1 < nch)
        def _():
            pltpu.make_async_copy(y_hbm.at[idxb1], buf1, g1).wait()
            consume(buf1, j + 1)

            @pl.when(j + 3 < nch)
            def _():
                fill_idx(idxb1, j + 3)
                pltpu.async_copy(y_hbm.at[idxb1], buf1, g1)

    @pl.when(wid < 16)
    def _():
        pltpu.sync_copy(acc_v.at[pl.ds(0, U_RPT * D)],
                        out_hbm.at[pl.ds(wid * U_RPT * D, U_RPT * D)])

    @pl.when(wid >= 16)
    def _():
        pltpu.sync_copy(
            acc_v.at[pl.ds(0, I_RPT * D)],
            out_hbm.at[pl.ds((NUM_USER + (wid - 16) * I_RPT) * D,
                             I_RPT * D)])


def _sc_conv(y, listr, listc, cnt, zeros_acc):
    kern = pl.kernel(
        _conv_body,
        mesh=_MESH,
        compiler_params=_CP,
        out_type=jax.ShapeDtypeStruct((N * D,), jnp.float32),
        scratch_types=[
            pltpu.VMEM((CAP,), jnp.int32),
            pltpu.VMEM((CAP,), jnp.int32),
            pltpu.VMEM((16,), jnp.int32),
            pltpu.VMEM((KB,), jnp.int32),
            pltpu.VMEM((KB,), jnp.int32),
            pltpu.VMEM((KB, D // 2), jnp.int32),
            pltpu.VMEM((KB, D // 2), jnp.int32),
            pltpu.VMEM((ACC_R * D,), jnp.float32),
            pltpu.SemaphoreType.DMA,
            pltpu.SemaphoreType.DMA,
        ],
    )
    return kern(y, listr, listc, cnt, zeros_acc)


def _ugraph_body(rep_hbm, ug_hbm, w_hbm, out_hbm, ug_v, w_v, idxb0, idxb1,
                 buf0, buf1, out_v, g0, g1):
    wid = _wid()
    pltpu.sync_copy(ug_hbm.at[pl.ds(wid * UPT * KU, UPT * KU)], ug_v)
    pltpu.sync_copy(w_hbm.at[pl.ds(wid * UPT * KU, UPT * KU)], w_v)

    def fill_idx(dst, u):
        for o in range(0, KU, 16):
            dst[pl.ds(o, 16)] = ug_v[pl.ds(u * KU + o, 16)]

    def consume(buf, uu):
        wvs = [w_v[pl.ds(uu * KU + ko, 16)] for ko in range(0, KU, 16)]
        for o in range(0, D // 2, 16):
            acc_lo = jnp.zeros((16,), jnp.float32)
            acc_hi = jnp.zeros((16,), jnp.float32)
            for k in range(KU):
                bv = plsc.bitcast(buf[k, pl.ds(o, 16)], jnp.bfloat16)
                lo, hi = plsc.unpack(bv, format=plsc.PackFormat.INTERLEAVED)
                wk = wvs[k // 16][k % 16]
                acc_lo = acc_lo + wk * lo
                acc_hi = acc_hi + wk * hi
            out_v[uu, pl.ds(2 * o, 16)] = acc_lo
            out_v[uu, pl.ds(2 * o + 16, 16)] = acc_hi

    fill_idx(idxb0, 0)
    pltpu.async_copy(rep_hbm.at[idxb0], buf0, g0)
    fill_idx(idxb1, 1)
    pltpu.async_copy(rep_hbm.at[idxb1], buf1, g1)

    @pl.loop(0, UPT, step=2)
    def _(u):
        pltpu.make_async_copy(rep_hbm.at[idxb0], buf0, g0).wait()
        consume(buf0, u)

        @pl.when(u + 2 < UPT)
        def _():
            fill_idx(idxb0, u + 2)
            pltpu.async_copy(rep_hbm.at[idxb0], buf0, g0)

        pltpu.make_async_copy(rep_hbm.at[idxb1], buf1, g1).wait()
        consume(buf1, u + 1)

        @pl.when(u + 3 < UPT)
        def _():
            fill_idx(idxb1, u + 3)
            pltpu.async_copy(rep_hbm.at[idxb1], buf1, g1)

    pltpu.sync_copy(out_v, out_hbm.at[pl.ds(wid * UPT, UPT)])


def _sc_ugraph(user_rep, ug_flat, w_flat):
    kern = pl.kernel(
        _ugraph_body,
        mesh=_MESH,
        compiler_params=_CP,
        out_type=jax.ShapeDtypeStruct((NW * UPT, D), jnp.float32),
        scratch_types=[
            pltpu.VMEM((UPT * KU,), jnp.int32),
            pltpu.VMEM((UPT * KU,), jnp.float32),
            pltpu.VMEM((KU,), jnp.int32),
            pltpu.VMEM((KU,), jnp.int32),
            pltpu.VMEM((KU, D // 2), jnp.int32),
            pltpu.VMEM((KU, D // 2), jnp.int32),
            pltpu.VMEM((UPT, D), jnp.float32),
            pltpu.SemaphoreType.DMA,
            pltpu.SemaphoreType.DMA,
        ],
    )
    return kern(user_rep, ug_flat, w_flat)


def _ileave_bf16(y):
    n = y.shape[0]
    yb = y.reshape(n, 8, 2, 16).swapaxes(2, 3).reshape(n, D // 2, 2) \
        .astype(jnp.bfloat16)
    return lax.bitcast_convert_type(yb, jnp.int32)


def _branch(temp, preference, lists, dis, zeros_acc):
    listr, listc, cnt = lists
    x = jnp.concatenate([preference, temp], axis=0)
    x = x / jnp.maximum(jnp.linalg.norm(x, axis=1, keepdims=True), 1e-12)
    disc = dis[:, None]
    y1 = x * disc
    s1 = _sc_conv(_ileave_bf16(y1), listr, listc, cnt,
                  zeros_acc).reshape(N, D)
    y2 = s1 * (disc * disc)
    s2 = _sc_conv(_ileave_bf16(y2), listr, listc, cnt,
                  zeros_acc).reshape(N, D)
    return x + (s1 + s2) * disc


def kernel(edge_index, v_feat, t_feat, pref_v, pref_t, W1v, b1v, W2v, b2v,
           W1t, b1t, W2t, b2t, weight_u, user_graph, user_weight_matrix):
    row = edge_index[0].astype(jnp.int32)
    col = edge_index[1].astype(jnp.int32)

    listr, listc, cnt, hist = _sc_prep(row, col)
    lists = (listr, listc, cnt)
    hsum = hist.reshape(NW, ACC_R, 16).sum(axis=2)
    deg = jnp.concatenate([hsum[:16, :U_RPT].reshape(NUM_USER),
                           hsum[16:, :I_RPT].reshape(NUM_ITEM)])
    dis = jnp.where(deg > 0, lax.rsqrt(jnp.maximum(deg, 1e-30)), 0.0)
    zeros_acc = jnp.zeros((ACC_R * D,), jnp.float32)

    temp_v = _mlp(v_feat, W1v, b1v, W2v, b2v)
    temp_t = _mlp(t_feat, W1t, b1t, W2t, b2t)
    v_rep = _branch(temp_v, pref_v, lists, dis, zeros_acc)
    t_rep = _branch(temp_t, pref_t, lists, dis, zeros_acc)
    representation = v_rep + t_rep
    user_stack = jnp.stack([v_rep[:NUM_USER], t_rep[:NUM_USER]], axis=2)
    user_rep = jnp.squeeze(user_stack @ weight_u, axis=2)
    item_rep = representation[NUM_USER:]

    ug = user_graph.astype(jnp.int32)
    ug = jnp.concatenate(
        [ug, jnp.zeros((NUM_USER, KU - ug.shape[1]), jnp.int32)], axis=1)
    ug = jnp.concatenate(
        [ug, jnp.zeros((NW * UPT - NUM_USER, KU), jnp.int32)], axis=0)
    w = jnp.concatenate(
        [user_weight_matrix,
         jnp.zeros((NUM_USER, KU - user_weight_matrix.shape[1]),
                   jnp.float32)], axis=1)
    w = jnp.concatenate(
        [w, jnp.zeros((NW * UPT - NUM_USER, KU), jnp.float32)], axis=0)
    h_u1 = _sc_ugraph(_ileave_bf16(user_rep), ug.reshape(-1),
                      w.reshape(-1))[:NUM_USER]
    user_rep = user_rep + h_u1
    return jnp.concatenate([user_rep, item_rep], axis=0)
